# baseline (speedup 1.0000x reference)
import jax
import jax.numpy as jnp
from jax import lax
from jax.experimental import pallas as pl
from jax.experimental.pallas import tpu as pltpu
from jax.experimental.pallas import tpu_sc as plsc

VOCAB = 100000
D_MODEL = 768
MAX_SEQ = 512
BATCH = 64
SEQ = 512

NC = 2
NS = 16
LANES = 16
NW = NC * NS
ROWS = BATCH * SEQ
ROWS_PER_W = ROWS // NW
S_PER_W = SEQ // NW
CHUNK = 32
NCHUNK = ROWS_PER_W // CHUNK
CPS = BATCH // CHUNK
NBUF = 4
DV = D_MODEL // LANES


def _body(ids_hbm, oidx_hbm, tok_hbm, pos_hbm, out_hbm,
          idx_v, oidx_v, pos_v, rows0, rows1, rows2, rows3,
          g0, g1, g2, g3, s0, s1, s2, s3):
    wid = lax.axis_index("s") * NC + lax.axis_index("c")
    base = wid * ROWS_PER_W
    pltpu.sync_copy(ids_hbm.at[pl.ds(base, ROWS_PER_W)], idx_v)
    pltpu.sync_copy(oidx_hbm.at[wid], oidx_v)
    pltpu.sync_copy(pos_hbm.at[pl.ds(wid * S_PER_W, S_PER_W)], pos_v)

    bufs = (rows0, rows1, rows2, rows3)
    gsems = (g0, g1, g2, g3)
    ssems = (s0, s1, s2, s3)

    def start_gather(j, buf, sem):
        pltpu.async_copy(tok_hbm.at[idx_v.at[pl.ds(j * CHUNK, CHUNK)]],
                         buf, sem)

    for jj in range(NBUF - 1):
        start_gather(jj, bufs[jj], gsems[jj])

    def group_step(p, carry):
        for b in range(NBUF):
            j = p * NBUF + b
            pre = (b + NBUF - 1) % NBUF

            @pl.when(j + NBUF - 1 < NCHUNK)
            def _prefetch():
                @pl.when(j >= 1)
                def _drain_store():
                    pltpu.make_async_copy(
                        bufs[pre], out_hbm.at[oidx_v.at[j - 1]],
                        ssems[pre]).wait()
                start_gather(j + NBUF - 1, bufs[pre], gsems[pre])

            pltpu.make_async_copy(
                tok_hbm.at[idx_v.at[pl.ds(j * CHUNK, CHUNK)]],
                bufs[b], gsems[b]).wait()

            pvecs = [pos_v[j // CPS, pl.ds(k * LANES, LANES)]
                     for k in range(DV)]

            def row_step(r, c):
                for k in range(DV):
                    sl = pl.ds(k * LANES, LANES)
                    bufs[b][r, sl] = bufs[b][r, sl] + pvecs[k]
                return c

            lax.fori_loop(0, CHUNK, row_step, 0)
            pltpu.async_copy(bufs[b], out_hbm.at[oidx_v.at[j]], ssems[b])
        return carry

    lax.fori_loop(0, NCHUNK // NBUF, group_step, 0)
    for jj in range(NBUF):
        j = NCHUNK - NBUF + jj
        pltpu.make_async_copy(bufs[j % NBUF], out_hbm.at[oidx_v.at[j]],
                              ssems[j % NBUF]).wait()


@jax.jit
def _run(ids_t, out_idx, token_table, position_table):
    mesh = plsc.VectorSubcoreMesh(core_axis_name="c", subcore_axis_name="s",
                                  num_cores=NC, num_subcores=NS)
    return pl.kernel(
        _body,
        out_type=jax.ShapeDtypeStruct((ROWS, D_MODEL), jnp.float32),
        mesh=mesh,
        scratch_types=[
            pltpu.VMEM((ROWS_PER_W,), jnp.int32),
            pltpu.VMEM((NCHUNK, CHUNK), jnp.int32),
            pltpu.VMEM((S_PER_W, D_MODEL), jnp.float32),
            pltpu.VMEM((CHUNK, D_MODEL), jnp.float32),
            pltpu.VMEM((CHUNK, D_MODEL), jnp.float32),
            pltpu.VMEM((CHUNK, D_MODEL), jnp.float32),
            pltpu.VMEM((CHUNK, D_MODEL), jnp.float32),
            pltpu.SemaphoreType.DMA,
            pltpu.SemaphoreType.DMA,
            pltpu.SemaphoreType.DMA,
            pltpu.SemaphoreType.DMA,
            pltpu.SemaphoreType.DMA,
            pltpu.SemaphoreType.DMA,
            pltpu.SemaphoreType.DMA,
            pltpu.SemaphoreType.DMA,
        ],
    )(ids_t, out_idx, token_table, position_table)


def kernel(input_ids, token_table, position_table):
    ids_t = input_ids.T.reshape(-1).astype(jnp.int32)
    s_ix = jnp.arange(SEQ, dtype=jnp.int32)
    b_ix = jnp.arange(BATCH, dtype=jnp.int32)
    out_idx = (b_ix[None, :] * SEQ + s_ix[:, None]).reshape(NW, NCHUNK, CHUNK)
    out = _run(ids_t, out_idx, token_table, position_table)
    return out.reshape(BATCH, SEQ, D_MODEL)


# speedup vs baseline: 1.0983x; 1.0983x over previous
import jax
import jax.numpy as jnp
from jax import lax
from jax.experimental import pallas as pl
from jax.experimental.pallas import tpu as pltpu
from jax.experimental.pallas import tpu_sc as plsc

VOCAB = 100000
D_MODEL = 768
MAX_SEQ = 512
BATCH = 64
SEQ = 512

NC = 2
NS = 16
LANES = 16
NW = NC * NS
ROWS = BATCH * SEQ
ROWS_PER_W = ROWS // NW
S_PER_W = SEQ // NW
CHUNK = 16
NCHUNK = ROWS_PER_W // CHUNK
CPS = BATCH // CHUNK
NBUF = 8
DV = D_MODEL // LANES


def _body(ids_hbm, oidx_hbm, tok_hbm, pos_hbm, out_hbm,
          idx_v, oidx_v, pos_v,
          rows0, rows1, rows2, rows3, rows4, rows5, rows6, rows7,
          g0, g1, g2, g3, g4, g5, g6, g7,
          s0, s1, s2, s3, s4, s5, s6, s7, stg):
    wid = lax.axis_index("s") * NC + lax.axis_index("c")
    base = wid * ROWS_PER_W
    pltpu.sync_copy(ids_hbm.at[pl.ds(base, ROWS_PER_W)], idx_v)

    bufs = (rows0, rows1, rows2, rows3, rows4, rows5, rows6, rows7)
    gsems = (g0, g1, g2, g3, g4, g5, g6, g7)
    ssems = (s0, s1, s2, s3, s4, s5, s6, s7)

    def start_gather(j, buf, sem):
        pltpu.async_copy(tok_hbm.at[idx_v.at[pl.ds(j * CHUNK, CHUNK)]],
                         buf, sem)

    for jj in range(NBUF - 1):
        start_gather(jj, bufs[jj], gsems[jj])
    pltpu.async_copy(oidx_hbm.at[wid], oidx_v, stg)
    pltpu.async_copy(pos_hbm.at[pl.ds(wid * S_PER_W, S_PER_W)], pos_v, stg)
    pltpu.make_async_copy(oidx_hbm.at[wid], oidx_v, stg).wait()
    pltpu.make_async_copy(pos_hbm.at[pl.ds(wid * S_PER_W, S_PER_W)],
                          pos_v, stg).wait()

    def group_step(p, carry):
        for b in range(NBUF):
            j = p * NBUF + b
            pre = (b + NBUF - 1) % NBUF

            @pl.when(j + NBUF - 1 < NCHUNK)
            def _prefetch():
                @pl.when(j >= 1)
                def _drain_store():
                    pltpu.make_async_copy(
                        bufs[pre], out_hbm.at[oidx_v.at[j - 1]],
                        ssems[pre]).wait()
                start_gather(j + NBUF - 1, bufs[pre], gsems[pre])

            pltpu.make_async_copy(
                tok_hbm.at[idx_v.at[pl.ds(j * CHUNK, CHUNK)]],
                bufs[b], gsems[b]).wait()

            pvecs = [pos_v[j // CPS, pl.ds(k * LANES, LANES)]
                     for k in range(DV)]

            def row_step(r, c):
                for k in range(DV):
                    sl = pl.ds(k * LANES, LANES)
                    plsc.addupdate(bufs[b].at[r, sl], pvecs[k])
                return c

            lax.fori_loop(0, CHUNK, row_step, 0)
            pltpu.async_copy(bufs[b], out_hbm.at[oidx_v.at[j]], ssems[b])
        return carry

    lax.fori_loop(0, NCHUNK // NBUF, group_step, 0)
    for jj in range(NBUF):
        j = NCHUNK - NBUF + jj
        pltpu.make_async_copy(bufs[j % NBUF], out_hbm.at[oidx_v.at[j]],
                              ssems[j % NBUF]).wait()


@jax.jit
def _run(ids_t, out_idx, token_table, position_table):
    mesh = plsc.VectorSubcoreMesh(core_axis_name="c", subcore_axis_name="s",
                                  num_cores=NC, num_subcores=NS)
    return pl.kernel(
        _body,
        out_type=jax.ShapeDtypeStruct((ROWS, D_MODEL), jnp.float32),
        mesh=mesh,
        scratch_types=(
            [pltpu.VMEM((ROWS_PER_W,), jnp.int32),
             pltpu.VMEM((NCHUNK, CHUNK), jnp.int32),
             pltpu.VMEM((S_PER_W, D_MODEL), jnp.float32)]
            + [pltpu.VMEM((CHUNK, D_MODEL), jnp.float32)] * NBUF
            + [pltpu.SemaphoreType.DMA] * (2 * NBUF + 1)
        ),
    )(ids_t, out_idx, token_table, position_table)


def kernel(input_ids, token_table, position_table):
    ids_t = input_ids.T.reshape(-1).astype(jnp.int32)
    s_ix = jnp.arange(SEQ, dtype=jnp.int32)
    b_ix = jnp.arange(BATCH, dtype=jnp.int32)
    out_idx = (b_ix[None, :] * SEQ + s_ix[:, None]).reshape(NW, NCHUNK, CHUNK)
    out = _run(ids_t, out_idx, token_table, position_table)
    return out.reshape(BATCH, SEQ, D_MODEL)
